# baseline (speedup 1.0000x reference)
import jax
import jax.numpy as jnp
from jax.experimental import pallas as pl

B = 128
C_IN = 256
EMD = 128
J = 21
FS = 32
P = FS * FS
JP = 32
NJ = 2 * JP


def _gather_body(uv_ref, img_ref, fl_ref, fr_ref):
    b = pl.program_id(0)
    uv = uv_ref[b]
    u = uv[0:1, :]
    v = uv[1:2, :]
    x = ((u + 1.0) * FS - 1.0) * 0.5
    y = ((v + 1.0) * FS - 1.0) * 0.5
    x0 = jnp.floor(x)
    y0 = jnp.floor(y)
    x1 = x0 + 1.0
    y1 = y0 + 1.0
    wx1 = x - x0
    wx0 = 1.0 - wx1
    wy1 = y - y0
    wy0 = 1.0 - wy1

    pio = jax.lax.broadcasted_iota(jnp.int32, (P, NJ), 0)

    def tap(ix, iy, w):
        valid = (ix >= 0.0) & (ix <= FS - 1.0) & (iy >= 0.0) & (iy <= FS - 1.0)
        wv = jnp.where(valid, w, 0.0)
        lin = (jnp.clip(iy, 0.0, FS - 1.0) * FS
               + jnp.clip(ix, 0.0, FS - 1.0)).astype(jnp.int32)
        return jnp.where(pio == lin, wv, 0.0)

    S = (tap(x0, y0, wx0 * wy0) + tap(x0, y1, wx0 * wy1)
         + tap(x1, y0, wx1 * wy0) + tap(x1, y1, wx1 * wy1))
    A = img_ref[0]
    feat = jax.lax.dot_general(A, S, (((1,), (0,)), ((), ())),
                               preferred_element_type=jnp.float32)
    fl_ref[:, 0, 0, :] = feat[:, 0:JP]
    fr_ref[:, 0, 0, :] = feat[:, JP:NJ]


def _head_body(feat_ref, w1_ref, b1_ref, g1_ref, be1_ref, w2_ref, b2_ref, out_ref):
    feat = feat_ref[...]
    h = jax.lax.dot_general(w1_ref[...], feat, (((1,), (0,)), ((), ())),
                            preferred_element_type=jnp.float32)
    h = h + b1_ref[...]
    col = jax.lax.broadcasted_iota(jnp.int32, (1, B * JP), 1)
    real = (col % JP) < J
    hm = jnp.where(real, h, 0.0)
    n = float(B * J)
    mean = jnp.sum(hm, axis=1, keepdims=True) * (1.0 / n)
    ex2 = jnp.sum(hm * hm, axis=1, keepdims=True) * (1.0 / n)
    var = ex2 - mean * mean
    hn = (h - mean) * jax.lax.rsqrt(var + 1e-5) * g1_ref[...] + be1_ref[...]
    hn = jnp.maximum(hn, 0.0)
    h2 = jax.lax.dot_general(w2_ref[...], hn, (((1,), (0,)), ((), ())),
                             preferred_element_type=jnp.float32)
    out_ref[...] = h2 + b2_ref[...]


def _head(feat, W1, b1, g1, be1, W2, b2):
    h2 = pl.pallas_call(
        _head_body,
        out_shape=jax.ShapeDtypeStruct((EMD, B * JP), jnp.float32),
    )(feat, W1, b1.reshape(EMD, 1), g1.reshape(EMD, 1),
      be1.reshape(EMD, 1), W2, b2.reshape(EMD, 1))
    return h2.reshape(EMD, B, JP).transpose(1, 2, 0)[:, :J, :]


def kernel(img_feat, joint_xyz_left, joint_xyz_right, joint_uv_left, joint_uv_right,
           pre_mano_para_left, pre_mano_para_right, offset,
           W1_l, b1_l, g1_l, be1_l, W2_l, b2_l,
           W1_r, b1_r, g1_r, be1_r, W2_r, b2_r):
    img = img_feat.reshape(B, C_IN, P)
    uv_l = jnp.pad(joint_uv_left, ((0, 0), (0, JP - J), (0, 0)))
    uv_r = jnp.pad(joint_uv_right, ((0, 0), (0, JP - J), (0, 0)))
    uv = jnp.concatenate([uv_l, uv_r], axis=1).transpose(0, 2, 1)

    feat_l, feat_r = pl.pallas_call(
        _gather_body,
        grid=(B,),
        in_specs=[
            pl.BlockSpec((B, 2, NJ), lambda b: (0, 0, 0)),
            pl.BlockSpec((1, C_IN, P), lambda b: (b, 0, 0)),
        ],
        out_specs=[
            pl.BlockSpec((C_IN, 1, 1, JP), lambda b: (0, b, 0, 0)),
            pl.BlockSpec((C_IN, 1, 1, JP), lambda b: (0, b, 0, 0)),
        ],
        out_shape=[
            jax.ShapeDtypeStruct((C_IN, B, 1, JP), jnp.float32),
            jax.ShapeDtypeStruct((C_IN, B, 1, JP), jnp.float32),
        ],
    )(uv, img)

    fl = _head(feat_l.reshape(C_IN, B * JP), W1_l, b1_l, g1_l, be1_l, W2_l, b2_l)
    fr = _head(feat_r.reshape(C_IN, B * JP), W1_r, b1_r, g1_r, be1_r, W2_r, b2_r)
    return (fl, fr)
